# Initial kernel scaffold; baseline (speedup 1.0000x reference)
import jax
import jax.numpy as jnp
from jax.experimental import pallas as pl


def kernel(latents, protos):
    raise NotImplementedError("write your pallas kernel here")


# speedup vs baseline: 1.4693x; 1.4693x over previous
import functools

import jax
import jax.numpy as jnp
from jax.experimental import pallas as pl
from jax.experimental.pallas import tpu as pltpu

_B, _G, _K, _D = 16384, 4, 512, 64
_BETA = 0.25
_BT = 512


def _vq_body(x_ref, p_ref, recon_ref, loss_ref, dists_ref):
    i = pl.program_id(0)

    @pl.when(i == 0)
    def _init():
        loss_ref[0, 0] = jnp.float32(0.0)

    acc = jnp.float32(0.0)
    for g in range(_G):
        x = x_ref[:, g, :]
        p = p_ref[g]
        xx = jnp.sum(x * x, axis=1, keepdims=True)
        pp = jnp.sum(p * p, axis=1)
        xp = jax.lax.dot_general(
            x, p, (((1,), (1,)), ((), ())),
            preferred_element_type=jnp.float32)
        d = xx - 2.0 * xp + pp[None, :]
        dists_ref[g] = d
        min_d = jnp.min(d, axis=1, keepdims=True)
        iota_k = jax.lax.broadcasted_iota(jnp.int32, (_BT, _K), 1)
        ind = jnp.min(jnp.where(d == min_d, iota_k, _K), axis=1)
        one_hot = (iota_k == ind[:, None]).astype(jnp.float32)
        q = jax.lax.dot_general(
            one_hot, p, (((1,), (0,)), ((), ())),
            preferred_element_type=jnp.float32,
            precision=jax.lax.Precision.HIGHEST)
        recon_ref[:, g, :] = q
        acc += jnp.sum(min_d)

    scale = jnp.float32((1.0 + _BETA) / (_G * _B * _D))
    loss_ref[0, 0] += acc * scale


@jax.jit
def kernel(latents, protos):
    n_tiles = _B // _BT
    recon, loss, dists = pl.pallas_call(
        _vq_body,
        grid=(n_tiles,),
        in_specs=[
            pl.BlockSpec((_BT, _G, _D), lambda i: (i, 0, 0)),
            pl.BlockSpec((_G, _K, _D), lambda i: (0, 0, 0)),
        ],
        out_specs=[
            pl.BlockSpec((_BT, _G, _D), lambda i: (i, 0, 0)),
            pl.BlockSpec(memory_space=pltpu.SMEM),
            pl.BlockSpec((_G, _BT, _K), lambda i: (0, i, 0)),
        ],
        out_shape=[
            jax.ShapeDtypeStruct((_B, _G, _D), jnp.float32),
            jax.ShapeDtypeStruct((1, 1), jnp.float32),
            jax.ShapeDtypeStruct((_G, _B, _K), jnp.float32),
        ],
    )(latents, protos)
    return recon, loss[0, 0], dists


# speedup vs baseline: 2.5082x; 1.7071x over previous
import jax
import jax.numpy as jnp
from jax.experimental import pallas as pl
from jax.experimental.pallas import tpu as pltpu

_B, _G, _K, _D = 16384, 4, 512, 64
_BETA = 0.25
_BT = 512


def _vq_body(x_ref, p_ref, recon_ref, loss_ref, dists_ref):
    i = pl.program_id(0)

    @pl.when(i == 0)
    def _init():
        loss_ref[0, 0] = jnp.float32(0.0)

    acc = jnp.float32(0.0)
    for g in range(_G):
        x = x_ref[:, g * _D:(g + 1) * _D]
        p = p_ref[g]
        xx = jnp.sum(x * x, axis=1, keepdims=True)
        pp = jnp.sum(p * p, axis=1)
        xp = jax.lax.dot_general(
            x, p, (((1,), (1,)), ((), ())),
            preferred_element_type=jnp.float32)
        d = xx - 2.0 * xp + pp[None, :]
        dists_ref[g] = d
        min_d = jnp.min(d, axis=1, keepdims=True)
        iota_k = jax.lax.broadcasted_iota(jnp.int32, (_BT, _K), 1)
        ind = jnp.min(jnp.where(d == min_d, iota_k, _K), axis=1)
        one_hot = (iota_k == ind[:, None]).astype(jnp.float32)
        q = jax.lax.dot_general(
            one_hot, p, (((1,), (0,)), ((), ())),
            preferred_element_type=jnp.float32)
        recon_ref[:, g * _D:(g + 1) * _D] = q
        acc += jnp.sum(min_d)

    scale = jnp.float32((1.0 + _BETA) / (_G * _B * _D))
    loss_ref[0, 0] += acc * scale


@jax.jit
def kernel(latents, protos):
    n_tiles = _B // _BT
    x2d = latents.reshape(_B, _G * _D)
    recon, loss, dists = pl.pallas_call(
        _vq_body,
        grid=(n_tiles,),
        in_specs=[
            pl.BlockSpec((_BT, _G * _D), lambda i: (i, 0)),
            pl.BlockSpec((_G, _K, _D), lambda i: (0, 0, 0)),
        ],
        out_specs=[
            pl.BlockSpec((_BT, _G * _D), lambda i: (i, 0)),
            pl.BlockSpec(memory_space=pltpu.SMEM),
            pl.BlockSpec((_G, _BT, _K), lambda i: (0, i, 0)),
        ],
        out_shape=[
            jax.ShapeDtypeStruct((_B, _G * _D), jnp.float32),
            jax.ShapeDtypeStruct((1, 1), jnp.float32),
            jax.ShapeDtypeStruct((_G, _B, _K), jnp.float32),
        ],
    )(x2d, protos)
    return recon.reshape(_B, _G, _D), loss[0, 0], dists
